# baseline (speedup 1.0000x reference)
import functools

import jax
import jax.numpy as jnp
from jax import lax
from jax.experimental import pallas as pl
from jax.experimental.pallas import tpu as pltpu
from jax.experimental.pallas import tpu_sc as plsc


def _silu(v):
    return v * jax.nn.sigmoid(v)


def _dot(a, b):
    return jnp.dot(a, b, preferred_element_type=jnp.float32)


def _pre_body(x_ref, rbf_ref, wji_ref, bji_ref, wkj_ref, bkj_ref, wrbf_ref,
              wdown_ref, xji_ref, t2_ref):
    halves = []
    for h in range(2):
        x = x_ref[h]
        xji_ref[h] = _silu(_dot(x, wji_ref[...]) + bji_ref[...])
        h_kj = _silu(_dot(x, wkj_ref[...]) + bkj_ref[...])
        h_kj = h_kj * _dot(rbf_ref[h], wrbf_ref[...])
        halves.append(_silu(_dot(h_kj, wdown_ref[...])))
    t2_ref[...] = jnp.concatenate(halves, axis=1)


def _stage_a(x3, rbf3, p, blk=2000):
    _, E2, EMB = x3.shape
    NRBF = rbf3.shape[2]
    INT = p['W_down'].shape[1]
    w_rbf = _dot(p['W_rbf1'], p['W_rbf2'])
    full = lambda shape: pl.BlockSpec(shape, lambda i: tuple(0 for _ in shape))
    return pl.pallas_call(
        _pre_body,
        grid=(E2 // blk,),
        in_specs=[
            pl.BlockSpec((2, blk, EMB), lambda i: (0, i, 0)),
            pl.BlockSpec((2, blk, NRBF), lambda i: (0, i, 0)),
            full((EMB, EMB)),
            full((1, EMB)),
            full((EMB, EMB)),
            full((1, EMB)),
            full((NRBF, EMB)),
            full((EMB, INT)),
        ],
        out_specs=[
            pl.BlockSpec((2, blk, EMB), lambda i: (0, i, 0)),
            pl.BlockSpec((blk, 2 * INT), lambda i: (i, 0)),
        ],
        out_shape=[
            jax.ShapeDtypeStruct((2, E2, EMB), jnp.float32),
            jax.ShapeDtypeStruct((E2, 2 * INT), jnp.float32),
        ],
    )(x3, rbf3, p['W_ji'], p['b_ji'][None, :], p['W_kj'], p['b_kj'][None, :],
      w_rbf, p['W_down'])


def _sbf_body(sbf_ref, w_ref, out_ref):
    out_ref[...] = jnp.concatenate(
        [_dot(sbf_ref[0], w_ref[...]), _dot(sbf_ref[1], w_ref[...])], axis=1)


def _stage_b(sbf3, p, blk=4000):
    _, A2, NSBF = sbf3.shape
    INT = p['W_sbf2'].shape[1]
    w_sbf = _dot(p['W_sbf1'], p['W_sbf2'])
    return pl.pallas_call(
        _sbf_body,
        grid=(A2 // blk,),
        in_specs=[
            pl.BlockSpec((2, blk, NSBF), lambda i: (0, i, 0)),
            pl.BlockSpec((NSBF, INT), lambda i: (0, 0)),
        ],
        out_specs=pl.BlockSpec((blk, 2 * INT), lambda i: (i, 0)),
        out_shape=jax.ShapeDtypeStruct((A2, 2 * INT), jnp.float32),
    )(sbf3, w_sbf)


def _post_body(pooled_ref, xji_ref, x_ref, wup_ref, wb0_ref, bb0_ref, wb1_ref,
               bb1_ref, wf_ref, bf_ref, wa_ref, ba_ref, out_ref):
    h = xji_ref[0] + _silu(_dot(pooled_ref[0], wup_ref[...]))
    th = _silu(_dot(h, wb0_ref[...]) + bb0_ref[...])
    h = h + _silu(_dot(th, wb1_ref[...]) + bb1_ref[...])
    h = _silu(_dot(h, wf_ref[...]) + bf_ref[...])
    o = x_ref[0] + h
    for i in range(2):
        t0 = _silu(_dot(o, wa_ref[i, 0]) + ba_ref[i, 0][None, :])
        t1 = _silu(_dot(t0, wa_ref[i, 1]) + ba_ref[i, 1][None, :])
        o = o + t1
    out_ref[0] = o


def _stage_d(pooled3, xji3, x3, p, blk=2000):
    _, E2, EMB = x3.shape
    INT = pooled3.shape[2]
    full = lambda shape: pl.BlockSpec(shape, lambda i, h: tuple(0 for _ in shape))
    return pl.pallas_call(
        _post_body,
        grid=(E2 // blk, 2),
        in_specs=[
            pl.BlockSpec((1, blk, INT), lambda i, h: (h, i, 0)),
            pl.BlockSpec((1, blk, EMB), lambda i, h: (h, i, 0)),
            pl.BlockSpec((1, blk, EMB), lambda i, h: (h, i, 0)),
            full((INT, EMB)),
            full((EMB, EMB)),
            full((1, EMB)),
            full((EMB, EMB)),
            full((1, EMB)),
            full((EMB, EMB)),
            full((1, EMB)),
            full((2, 2, EMB, EMB)),
            full((2, 2, EMB)),
        ],
        out_specs=pl.BlockSpec((1, blk, EMB), lambda i, h: (h, i, 0)),
        out_shape=jax.ShapeDtypeStruct((2, E2, EMB), jnp.float32),
    )(pooled3, xji3, x3, p['W_up'],
      p['res_before_W'][0, 0], p['res_before_b'][0, 0][None, :],
      p['res_before_W'][0, 1], p['res_before_b'][0, 1][None, :],
      p['W_final'], p['b_final'][None, :],
      p['res_after_W'], p['res_after_b'])


_NC, _NS = 2, 16
_D = 64
_G = 64
_ZR = 40
_C = 1600


def _sc_middle_call(E, A, lens):
    E2, A2 = E // 2, A // 2
    AW = A // _NS
    NCH = AW // _C
    CAP = ((_C + _G - 1) // _G) * _G
    offs = [sum(lens[:i]) for i in range(len(lens))]
    mo8 = lambda v: pl.multiple_of(v, 8)

    def body(t_hbm, sbf_hbm, id0_hbm, id1_hbm, out_hbm,
             id0c, id1c, tixs, sixs, aixs2, pqs, tixb, sixb,
             trows0, trows1, srows0, srows1, prod0, prod1, zbuf, acc,
             sem_gt0, sem_gt1, sem_gs0, sem_gs1, sem_s0, sem_s1):
        c = lax.axis_index("c")
        s = lax.axis_index("s")
        trows = (trows0, trows1)
        srows = (srows0, srows1)
        prod = (prod0, prod1)
        sem_gt = (sem_gt0, sem_gt1)
        sem_gs = (sem_gs0, sem_gs1)
        sem_s = (sem_s0, sem_s1)

        def zb_init(z, carry):
            for q in range(4):
                zbuf[z, pl.ds(q * 16, 16)] = jnp.zeros((16,), jnp.float32)
            return carry
        lax.fori_loop(0, _ZR, zb_init, 0)

        def wait_gathers(par):
            pltpu.make_async_copy(t_hbm.at[tixb.at[par]], trows[par],
                                  sem_gt[par]).wait()
            pltpu.make_async_copy(sbf_hbm.at[sixb.at[par]], srows[par],
                                  sem_gs[par]).wait()


        def build_fire(b, par):
            for g in range(_G // 16):
                off = b * _G + g * 16
                tixb[par, pl.ds(g * 16, 16)] = tixs[pl.ds(off, 16)]
                sixb[par, pl.ds(g * 16, 16)] = sixs[pl.ds(off, 16)]
            pltpu.async_copy(t_hbm.at[tixb.at[par]], trows[par], sem_gt[par])
            pltpu.async_copy(sbf_hbm.at[sixb.at[par]], srows[par],
                             sem_gs[par])

        def multiply(b, par):
            def mgrp(gi, carry):
                codes = pqs[pl.ds(b * _G + gi * 16, 16)]
                for rr in range(16):
                    r = gi * 16 + rr
                    code = codes[rr]
                    toff = code & 0xFFFF
                    soff = code >> 16
                    for q in range(4):
                        prod[par][r, pl.ds(q * 16, 16)] = (
                            trows[par][r, pl.ds(toff + q * 16, 16)]
                            * srows[par][r, pl.ds(soff + q * 16, 16)])
                return carry
            lax.fori_loop(0, _G // 16, mgrp, 0)

        TRASH = 12800

        def do_range(lo, L, RS):
            def zc(j, carry):
                pltpu.sync_copy(zbuf, acc.at[pl.ds(mo8(s * RS + j * _ZR),
                                                   _ZR)])
                return carry
            lax.fori_loop(0, RS // _ZR, zc, 0)
            plsc.subcore_barrier()

            def do_chunk(ch, carry_c):
                abase = mo8(s * AW + ch * _C)
                pltpu.sync_copy(id0_hbm.at[pl.ds(abase, _C)], id0c)
                pltpu.sync_copy(id1_hbm.at[pl.ds(abase, _C)], id1c)

                def scan_g(g, n):
                    iota16 = lax.iota(jnp.int32, 16)
                    a = abase + g * 16 + iota16
                    v0 = id0c[pl.ds(g * 16, 16)]
                    v1 = id1c[pl.ds(g * 16, 16)]
                    m = (v0 >= lo) & (v0 < lo + L)
                    mi = jnp.where(m, 1, 0)
                    pos = n + plsc.cumsum(mi) - mi
                    hi_t = v1 >= E2
                    hi_s = a >= A2
                    code = (jnp.where(hi_t, 64, 0)
                            | jnp.where(hi_s, 64 << 16, 0))
                    plsc.store_scatter(tixs, [pos],
                                       jnp.where(hi_t, v1 - E2, v1), mask=m)
                    plsc.store_scatter(sixs, [pos],
                                       jnp.where(hi_s, a - A2, a), mask=m)
                    plsc.store_scatter(aixs2, [pos >> 6, pos & 63],
                                       v0 - lo, mask=m)
                    plsc.store_scatter(pqs, [pos], code, mask=m)
                    return n + jnp.sum(mi)
                n = lax.fori_loop(0, _C // 16, scan_g, jnp.int32(0))
                nb = (n + _G - 1) // _G

                def padg(g, carry):
                    iota16 = lax.iota(jnp.int32, 16)
                    off = g * 16 + iota16
                    mk = off >= n
                    zi = jnp.zeros((16,), jnp.int32)
                    plsc.store_scatter(tixs, [off], zi, mask=mk)
                    plsc.store_scatter(sixs, [off], zi, mask=mk)
                    plsc.store_scatter(aixs2, [off >> 6, off & 63],
                                       TRASH + iota16, mask=mk)
                    plsc.store_scatter(pqs, [off], zi, mask=mk)
                    return carry
                lax.fori_loop(n // 16, nb * (_G // 16), padg, 0)

                def batch_body(b, carry):
                    tsl = tixs.at[pl.ds(b * _G, _G)]
                    ssl = sixs.at[pl.ds(b * _G, _G)]
                    pltpu.async_copy(t_hbm.at[tsl], trows0, sem_gt0).wait()
                    pltpu.async_copy(sbf_hbm.at[ssl], srows0, sem_gs0).wait()
                    _DBG_MS = 2
                    if _DBG_MS >= 1:
                        multiply(b, 0)
                    if _DBG_MS >= 2:
                        pltpu.async_copy(prod[0], acc.at[aixs2.at[b]],
                                         sem_s0, add=True).wait()
                    return carry
                lax.fori_loop(0, nb, batch_body, 0)
                return carry_c
            lax.fori_loop(0, NCH, do_chunk, 0)

            plsc.subcore_barrier()
            pltpu.sync_copy(acc.at[pl.ds(mo8(s * RS), RS)],
                            out_hbm.at[pl.ds(mo8(lo + s * RS), RS)])

        LU = lens[0]
        NPU = len(lens) - 1
        LE = lens[-1]

        def do_pass(p, carry_p):
            do_range((p * _NC + c) * LU, LU, LU // _NS)
            return carry_p
        lax.fori_loop(0, NPU, do_pass, 0)
        do_range(NPU * _NC * LU + c * LE, LE, LE // _NS)

    return pl.kernel(
        body,
        out_type=jax.ShapeDtypeStruct((E, _D), jnp.float32),
        compiler_params=pltpu.CompilerParams(needs_layout_passes=False, use_tc_tiling_on_sc=False),
        mesh=plsc.VectorSubcoreMesh(core_axis_name="c", subcore_axis_name="s", num_cores=_NC, num_subcores=_NS),
        scratch_types=[
            pltpu.VMEM((_C,), jnp.int32),
            pltpu.VMEM((_C,), jnp.int32),
            pltpu.VMEM((CAP,), jnp.int32),
            pltpu.VMEM((CAP,), jnp.int32),
            pltpu.VMEM((CAP // _G, _G), jnp.int32),
            pltpu.VMEM((CAP,), jnp.int32),
            pltpu.VMEM((2, _G), jnp.int32),
            pltpu.VMEM((2, _G), jnp.int32),
            pltpu.VMEM((_G, 2 * _D), jnp.float32),
            pltpu.VMEM((_G, 2 * _D), jnp.float32),
            pltpu.VMEM((_G, 2 * _D), jnp.float32),
            pltpu.VMEM((_G, 2 * _D), jnp.float32),
            pltpu.VMEM((_G, _D), jnp.float32),
            pltpu.VMEM((_G, _D), jnp.float32),
            pltpu.VMEM((_ZR, _D), jnp.float32),
            pltpu.VMEM_SHARED((16016, _D), jnp.float32),
            pltpu.SemaphoreType.DMA,
            pltpu.SemaphoreType.DMA,
            pltpu.SemaphoreType.DMA,
            pltpu.SemaphoreType.DMA,
            pltpu.SemaphoreType.DMA,
            pltpu.SemaphoreType.DMA,
        ],
    )


def _sc_middle(t2, sbf2, id0, id1):
    E = t2.shape[0] * 2
    A = sbf2.shape[0] * 2
    L = 12800
    npu = (E // 2) // L
    le = E // 2 - npu * L
    if le == 0:
        npu -= 1
        le = L
    for l in (L, le):
        assert l > 0 and l % (_NS * _ZR) == 0 and (l // _NS) % 8 == 0
    lens = [L] * npu + [le]
    assert (A // _NS) % _C == 0
    return _sc_middle_call(E, A, tuple(lens))(t2, sbf2, id0, id1)


def kernel(x, rbf, sbf, id_expand, params):
    E, EMB = x.shape
    A, NSBF = sbf.shape
    E2, A2 = E // 2, A // 2
    x3 = x.reshape(2, E2, EMB)
    rbf3 = rbf.reshape(2, E2, rbf.shape[1])
    sbf3 = sbf.reshape(2, A2, NSBF)
    xji3, t2 = _stage_a(x3, rbf3, params)
    sbf2 = _stage_b(sbf3, params)
    pooled = _sc_middle(t2, sbf2, id_expand[:, 0], id_expand[:, 1])
    pooled3 = pooled.reshape(2, E2, pooled.shape[1])
    out3 = _stage_d(pooled3, xji3, x3, params)
    return out3.reshape(E, EMB)


# speedup vs baseline: 1.0449x; 1.0449x over previous
import functools

import jax
import jax.numpy as jnp
from jax import lax
from jax.experimental import pallas as pl
from jax.experimental.pallas import tpu as pltpu
from jax.experimental.pallas import tpu_sc as plsc


def _silu(v):
    return v * jax.nn.sigmoid(v)


def _dot(a, b):
    return jnp.dot(a, b, preferred_element_type=jnp.float32)


def _pre_body(x_ref, rbf_ref, wji_ref, bji_ref, wkj_ref, bkj_ref, wrbf_ref,
              wdown_ref, xji_ref, t2_ref):
    halves = []
    for h in range(2):
        x = x_ref[h]
        xji_ref[h] = _silu(_dot(x, wji_ref[...]) + bji_ref[...])
        h_kj = _silu(_dot(x, wkj_ref[...]) + bkj_ref[...])
        h_kj = h_kj * _dot(rbf_ref[h], wrbf_ref[...])
        halves.append(_silu(_dot(h_kj, wdown_ref[...])))
    t2_ref[...] = jnp.concatenate(halves, axis=1)


def _stage_a(x3, rbf3, p, blk=2000):
    _, E2, EMB = x3.shape
    NRBF = rbf3.shape[2]
    INT = p['W_down'].shape[1]
    w_rbf = _dot(p['W_rbf1'], p['W_rbf2'])
    full = lambda shape: pl.BlockSpec(shape, lambda i: tuple(0 for _ in shape))
    return pl.pallas_call(
        _pre_body,
        grid=(E2 // blk,),
        in_specs=[
            pl.BlockSpec((2, blk, EMB), lambda i: (0, i, 0)),
            pl.BlockSpec((2, blk, NRBF), lambda i: (0, i, 0)),
            full((EMB, EMB)),
            full((1, EMB)),
            full((EMB, EMB)),
            full((1, EMB)),
            full((NRBF, EMB)),
            full((EMB, INT)),
        ],
        out_specs=[
            pl.BlockSpec((2, blk, EMB), lambda i: (0, i, 0)),
            pl.BlockSpec((blk, 2 * INT), lambda i: (i, 0)),
        ],
        out_shape=[
            jax.ShapeDtypeStruct((2, E2, EMB), jnp.float32),
            jax.ShapeDtypeStruct((E2, 2 * INT), jnp.float32),
        ],
    )(x3, rbf3, p['W_ji'], p['b_ji'][None, :], p['W_kj'], p['b_kj'][None, :],
      w_rbf, p['W_down'])


def _sbf_body(sbf_ref, w_ref, out_ref):
    out_ref[...] = jnp.concatenate(
        [_dot(sbf_ref[0], w_ref[...]), _dot(sbf_ref[1], w_ref[...])], axis=1)


def _stage_b(sbf3, p, blk=4000):
    _, A2, NSBF = sbf3.shape
    INT = p['W_sbf2'].shape[1]
    w_sbf = _dot(p['W_sbf1'], p['W_sbf2'])
    return pl.pallas_call(
        _sbf_body,
        grid=(A2 // blk,),
        in_specs=[
            pl.BlockSpec((2, blk, NSBF), lambda i: (0, i, 0)),
            pl.BlockSpec((NSBF, INT), lambda i: (0, 0)),
        ],
        out_specs=pl.BlockSpec((blk, 2 * INT), lambda i: (i, 0)),
        out_shape=jax.ShapeDtypeStruct((A2, 2 * INT), jnp.float32),
    )(sbf3, w_sbf)


def _post_body(pooled_ref, xji_ref, x_ref, wup_ref, wb0_ref, bb0_ref, wb1_ref,
               bb1_ref, wf_ref, bf_ref, wa_ref, ba_ref, out_ref):
    h = xji_ref[0] + _silu(_dot(pooled_ref[0], wup_ref[...]))
    th = _silu(_dot(h, wb0_ref[...]) + bb0_ref[...])
    h = h + _silu(_dot(th, wb1_ref[...]) + bb1_ref[...])
    h = _silu(_dot(h, wf_ref[...]) + bf_ref[...])
    o = x_ref[0] + h
    for i in range(2):
        t0 = _silu(_dot(o, wa_ref[i, 0]) + ba_ref[i, 0][None, :])
        t1 = _silu(_dot(t0, wa_ref[i, 1]) + ba_ref[i, 1][None, :])
        o = o + t1
    out_ref[0] = o


def _stage_d(pooled3, xji3, x3, p, blk=2000):
    _, E2, EMB = x3.shape
    INT = pooled3.shape[2]
    full = lambda shape: pl.BlockSpec(shape, lambda i, h: tuple(0 for _ in shape))
    return pl.pallas_call(
        _post_body,
        grid=(E2 // blk, 2),
        in_specs=[
            pl.BlockSpec((1, blk, INT), lambda i, h: (h, i, 0)),
            pl.BlockSpec((1, blk, EMB), lambda i, h: (h, i, 0)),
            pl.BlockSpec((1, blk, EMB), lambda i, h: (h, i, 0)),
            full((INT, EMB)),
            full((EMB, EMB)),
            full((1, EMB)),
            full((EMB, EMB)),
            full((1, EMB)),
            full((EMB, EMB)),
            full((1, EMB)),
            full((2, 2, EMB, EMB)),
            full((2, 2, EMB)),
        ],
        out_specs=pl.BlockSpec((1, blk, EMB), lambda i, h: (h, i, 0)),
        out_shape=jax.ShapeDtypeStruct((2, E2, EMB), jnp.float32),
    )(pooled3, xji3, x3, p['W_up'],
      p['res_before_W'][0, 0], p['res_before_b'][0, 0][None, :],
      p['res_before_W'][0, 1], p['res_before_b'][0, 1][None, :],
      p['W_final'], p['b_final'][None, :],
      p['res_after_W'], p['res_after_b'])


_NC, _NS = 2, 16
_D = 64
_G = 64
_ZR = 40
_C = 1600


def _sc_middle_call(E, A, lens):
    E2, A2 = E // 2, A // 2
    AW = A // _NS
    NCH = AW // _C
    CAP = ((_C + _G - 1) // _G) * _G
    offs = [sum(lens[:i]) for i in range(len(lens))]
    mo8 = lambda v: pl.multiple_of(v, 8)

    def body(t_hbm, sbf_hbm, id0_hbm, id1_hbm, out_hbm,
             id0c0, id0c1, id1c0, id1c1, tixs, sixs, aixs2, pqs,
             trows0, trows1, srows0, srows1, prod0, prod1, zbuf, acc,
             sem_gt0, sem_gt1, sem_gs0, sem_gs1, sem_s0, sem_s1,
             sem_id0, sem_id1):
        c = lax.axis_index("c")
        s = lax.axis_index("s")
        id0c = (id0c0, id0c1)
        id1c = (id1c0, id1c1)
        trows = (trows0, trows1)
        srows = (srows0, srows1)
        prod = (prod0, prod1)
        sem_gt = (sem_gt0, sem_gt1)
        sem_gs = (sem_gs0, sem_gs1)
        sem_s = (sem_s0, sem_s1)
        sem_id = (sem_id0, sem_id1)

        def zb_init(z, carry):
            for q in range(4):
                zbuf[z, pl.ds(q * 16, 16)] = jnp.zeros((16,), jnp.float32)
            return carry
        lax.fori_loop(0, _ZR, zb_init, 0)

        def fire_gathers(b, par):
            pltpu.async_copy(t_hbm.at[tixs.at[pl.ds(b * _G, _G)]],
                             trows[par], sem_gt[par])
            pltpu.async_copy(sbf_hbm.at[sixs.at[pl.ds(b * _G, _G)]],
                             srows[par], sem_gs[par])

        def wait_gathers(b, par):
            pltpu.make_async_copy(t_hbm.at[tixs.at[pl.ds(b * _G, _G)]],
                                  trows[par], sem_gt[par]).wait()
            pltpu.make_async_copy(sbf_hbm.at[sixs.at[pl.ds(b * _G, _G)]],
                                  srows[par], sem_gs[par]).wait()

        def fire_scatter(b, par):
            pltpu.async_copy(prod[par], acc.at[aixs2.at[b]], sem_s[par],
                             add=True)

        def wait_scatter(b, par):
            pltpu.make_async_copy(prod[par], acc.at[aixs2.at[b]],
                                  sem_s[par]).wait()

        def fire_ids(ch, parc):
            abase = mo8(s * AW + ch * _C)
            pltpu.async_copy(id0_hbm.at[pl.ds(abase, _C)], id0c[parc],
                             sem_id[parc])
            pltpu.async_copy(id1_hbm.at[pl.ds(abase, _C)], id1c[parc],
                             sem_id[parc])

        def wait_ids(ch, parc):
            abase = mo8(s * AW + ch * _C)
            pltpu.make_async_copy(id0_hbm.at[pl.ds(abase, _C)], id0c[parc],
                                  sem_id[parc]).wait()
            pltpu.make_async_copy(id1_hbm.at[pl.ds(abase, _C)], id1c[parc],
                                  sem_id[parc]).wait()

        def multiply(b, par):
            def mgrp(gi, carry):
                codes = pqs[pl.ds(b * _G + gi * 16, 16)]
                for rr in range(16):
                    r = gi * 16 + rr
                    code = codes[rr]
                    toff = code & 0xFFFF
                    soff = code >> 16
                    for q in range(4):
                        prod[par][r, pl.ds(q * 16, 16)] = (
                            trows[par][r, pl.ds(toff + q * 16, 16)]
                            * srows[par][r, pl.ds(soff + q * 16, 16)])
                return carry
            lax.fori_loop(0, _G // 16, mgrp, 0)

        TRASH = 12800

        def do_range(lo, L, RS):
            def zc(j, carry):
                pltpu.sync_copy(zbuf, acc.at[pl.ds(mo8(s * RS + j * _ZR),
                                                   _ZR)])
                return carry
            lax.fori_loop(0, RS // _ZR, zc, 0)
            plsc.subcore_barrier()

            def do_chunk(ch, parc):
                abase = mo8(s * AW + ch * _C)
                wait_ids(ch, parc)
                @pl.when(ch + 1 < NCH)
                def _():
                    fire_ids(ch + 1, 1 - parc)

                def scan_g(g, n):
                    iota16 = lax.iota(jnp.int32, 16)
                    a = abase + g * 16 + iota16
                    v0 = id0c[parc][pl.ds(g * 16, 16)]
                    v1 = id1c[parc][pl.ds(g * 16, 16)]
                    m = (v0 >= lo) & (v0 < lo + L)
                    mi = jnp.where(m, 1, 0)
                    pos = n + plsc.cumsum(mi) - mi
                    hi_t = v1 >= E2
                    hi_s = a >= A2
                    code = (jnp.where(hi_t, 64, 0)
                            | jnp.where(hi_s, 64 << 16, 0))
                    plsc.store_scatter(tixs, [pos],
                                       jnp.where(hi_t, v1 - E2, v1), mask=m)
                    plsc.store_scatter(sixs, [pos],
                                       jnp.where(hi_s, a - A2, a), mask=m)
                    plsc.store_scatter(aixs2, [pos >> 6, pos & 63],
                                       v0 - lo, mask=m)
                    plsc.store_scatter(pqs, [pos], code, mask=m)
                    return n + jnp.sum(mi)
                n = lax.fori_loop(0, _C // 16, scan_g, jnp.int32(0))
                nb = (n + _G - 1) // _G

                def padg(g, carry):
                    iota16 = lax.iota(jnp.int32, 16)
                    off = g * 16 + iota16
                    mk = off >= n
                    zi = jnp.zeros((16,), jnp.int32)
                    plsc.store_scatter(tixs, [off], zi, mask=mk)
                    plsc.store_scatter(sixs, [off], zi, mask=mk)
                    plsc.store_scatter(aixs2, [off >> 6, off & 63],
                                       TRASH + iota16, mask=mk)
                    plsc.store_scatter(pqs, [off], zi, mask=mk)
                    return carry
                lax.fori_loop(n // 16, nb * (_G // 16), padg, 0)

                @pl.when(nb > 0)
                def _():
                    fire_gathers(jnp.int32(0), 0)

                def pair_body(i, carry):
                    for par in range(2):
                        b = 2 * i + par
                        @pl.when(b < nb)
                        def _():
                            wait_gathers(b, par)
                            @pl.when(b + 1 < nb)
                            def _():
                                fire_gathers(b + 1, 1 - par)
                            multiply(b, par)
                            fire_scatter(b, par)
                            wait_scatter(b, par)
                    return carry
                lax.fori_loop(0, (nb + 1) // 2, pair_body, 0)
            def chunk_pair(i, carry):
                for parc in range(2):
                    ch = 2 * i + parc
                    @pl.when(ch < NCH)
                    def _():
                        do_chunk(ch, parc)
                return carry
            fire_ids(jnp.int32(0), 0)
            lax.fori_loop(0, (NCH + 1) // 2, chunk_pair, 0)

            plsc.subcore_barrier()
            pltpu.sync_copy(acc.at[pl.ds(mo8(s * RS), RS)],
                            out_hbm.at[pl.ds(mo8(lo + s * RS), RS)])

        LU = lens[0]
        NPU = len(lens) - 1
        LE = lens[-1]

        def do_pass(p, carry_p):
            do_range((p * _NC + c) * LU, LU, LU // _NS)
            return carry_p
        lax.fori_loop(0, NPU, do_pass, 0)
        do_range(NPU * _NC * LU + c * LE, LE, LE // _NS)

    return pl.kernel(
        body,
        out_type=jax.ShapeDtypeStruct((E, _D), jnp.float32),
        compiler_params=pltpu.CompilerParams(needs_layout_passes=False, use_tc_tiling_on_sc=False),
        mesh=plsc.VectorSubcoreMesh(core_axis_name="c", subcore_axis_name="s", num_cores=_NC, num_subcores=_NS),
        scratch_types=[
            pltpu.VMEM((_C,), jnp.int32),
            pltpu.VMEM((_C,), jnp.int32),
            pltpu.VMEM((_C,), jnp.int32),
            pltpu.VMEM((_C,), jnp.int32),
            pltpu.VMEM((CAP,), jnp.int32),
            pltpu.VMEM((CAP,), jnp.int32),
            pltpu.VMEM((CAP // _G, _G), jnp.int32),
            pltpu.VMEM((CAP,), jnp.int32),
            pltpu.VMEM((_G, 2 * _D), jnp.float32),
            pltpu.VMEM((_G, 2 * _D), jnp.float32),
            pltpu.VMEM((_G, 2 * _D), jnp.float32),
            pltpu.VMEM((_G, 2 * _D), jnp.float32),
            pltpu.VMEM((_G, _D), jnp.float32),
            pltpu.VMEM((_G, _D), jnp.float32),
            pltpu.VMEM((_ZR, _D), jnp.float32),
            pltpu.VMEM_SHARED((16016, _D), jnp.float32),
            pltpu.SemaphoreType.DMA,
            pltpu.SemaphoreType.DMA,
            pltpu.SemaphoreType.DMA,
            pltpu.SemaphoreType.DMA,
            pltpu.SemaphoreType.DMA,
            pltpu.SemaphoreType.DMA,
            pltpu.SemaphoreType.DMA,
            pltpu.SemaphoreType.DMA,
        ],
    )


def _sc_middle(t2, sbf2, id0, id1):
    E = t2.shape[0] * 2
    A = sbf2.shape[0] * 2
    L = 12800
    npu = (E // 2) // L
    le = E // 2 - npu * L
    if le == 0:
        npu -= 1
        le = L
    for l in (L, le):
        assert l > 0 and l % (_NS * _ZR) == 0 and (l // _NS) % 8 == 0
    lens = [L] * npu + [le]
    assert (A // _NS) % _C == 0
    return _sc_middle_call(E, A, tuple(lens))(t2, sbf2, id0, id1)


def kernel(x, rbf, sbf, id_expand, params):
    E, EMB = x.shape
    A, NSBF = sbf.shape
    E2, A2 = E // 2, A // 2
    x3 = x.reshape(2, E2, EMB)
    rbf3 = rbf.reshape(2, E2, rbf.shape[1])
    sbf3 = sbf.reshape(2, A2, NSBF)
    xji3, t2 = _stage_a(x3, rbf3, params)
    sbf2 = _stage_b(sbf3, params)
    pooled = _sc_middle(t2, sbf2, id_expand[:, 0], id_expand[:, 1])
    pooled3 = pooled.reshape(2, E2, pooled.shape[1])
    out3 = _stage_d(pooled3, xji3, x3, params)
    return out3.reshape(E, EMB)
